# baseline (speedup 1.0000x reference)
import functools

import jax
import jax.numpy as jnp
from jax import lax
from jax.experimental import pallas as pl
from jax.experimental.pallas import tpu as pltpu
from jax.experimental.pallas import tpu_sc as plsc

NC = 2
NS = 16
NW = NC * NS
L = 16
TD = 128


@functools.lru_cache(maxsize=None)
def _make_sc_kernel(B, K, D, V):
    RB = B // NW
    NBUF = 4
    assert B % NW == 0 and RB % NBUF == 0 and K % 8 == 0
    bases = list(range(0, K, L))
    if bases[-1] + L > K:
        bases[-1] = K - L

    mesh = plsc.VectorSubcoreMesh(core_axis_name="c", subcore_axis_name="s")

    @functools.partial(
        pl.kernel,
        out_type=jax.ShapeDtypeStruct((B * K,), jnp.float32),
        mesh=mesh,
        compiler_params=pltpu.CompilerParams(
            use_tc_tiling_on_sc=False, needs_layout_passes=False),
        scratch_types=[
            pltpu.VMEM((RB * K,), jnp.int32),
            pltpu.VMEM((RB, D), jnp.float32),
            [pltpu.VMEM((K, D), jnp.float32) for _ in range(NBUF)],
            pltpu.VMEM((RB * K,), jnp.float32),
            [pltpu.SemaphoreType.DMA for _ in range(NBUF)],
        ],
    )
    def sc_kernel(z_hbm, idx_hbm, table_hbm, out_hbm,
                  idx_v, z_v, rows, out_v, sems):
        wid = lax.axis_index("s") * NC + lax.axis_index("c")
        base = wid * RB
        pltpu.sync_copy(idx_hbm.at[pl.ds(base * K, RB * K)], idx_v)
        pltpu.sync_copy(z_hbm.at[pl.ds(base, RB)], z_v)

        def issue(i, rows_buf, sem):
            pltpu.async_copy(
                table_hbm.at[idx_v.at[pl.ds(i * K, K)]], rows_buf, sem)

        def wait(rows_buf, sem):
            pltpu.make_async_copy(
                table_hbm.at[pl.ds(0, K)], rows_buf, sem).wait()

        iota = lax.iota(jnp.int32, L)

        def compute(i, rows_buf):
            ivec = jnp.broadcast_to(i, (L,))
            nh = len(bases) // 2 + 1
            for half in (bases[:nh], bases[nh:]):
                def rbody(r, accs, half=half):
                    rot = (iota + r) & (L - 1)
                    new = list(accs)
                    for dg in range(D // L):
                        dvec = dg * L + rot
                        m = plsc.load_gather(z_v, [ivec, dvec])
                        for gi, kb in enumerate(half):
                            col = plsc.load_gather(
                                rows_buf, [kb + iota, dvec])
                            new[gi] = new[gi] + m * col
                    return tuple(new)
                accs = lax.fori_loop(
                    0, L, rbody,
                    tuple(jnp.zeros((L,), jnp.float32) for _ in half))
                for gi, kb in enumerate(half):
                    out_v[pl.ds(i * K + kb, L)] = accs[gi]

        for s in range(NBUF):
            issue(s, rows[s], sems[s])

        def body(j, carry):
            for s in range(NBUF):
                i = NBUF * j + s
                wait(rows[s], sems[s])
                compute(i, rows[s])

                @pl.when(i + NBUF < RB)
                def _(s=s, i=i):
                    issue(i + NBUF, rows[s], sems[s])
            return carry

        lax.fori_loop(0, RB // NBUF, body, 0)
        pltpu.sync_copy(out_v, out_hbm.at[pl.ds(base * K, RB * K)])

    return sc_kernel


def kernel(z_B1D, cand_idx_BK, id_embed):
    B, _, D = z_B1D.shape
    K = cand_idx_BK.shape[1]
    V = id_embed.shape[0]
    z = z_B1D.reshape(B, D)
    idx = cand_idx_BK.reshape(-1)
    if idx.dtype != jnp.int32:
        idx = idx.astype(jnp.int32)
    table = jnp.pad(id_embed.T, ((0, TD - D), (0, 0))).T.reshape(2 * V, D)
    idx2 = idx * 2
    out = _make_sc_kernel(B, K, D, V)(z, idx2, table)
    return out.reshape(B, K)


# speedup vs baseline: 1.1117x; 1.1117x over previous
import functools

import jax
import jax.numpy as jnp
from jax import lax
from jax.experimental import pallas as pl
from jax.experimental.pallas import tpu as pltpu
from jax.experimental.pallas import tpu_sc as plsc

NC = 2
NS = 16
NW = NC * NS
L = 16
TD = 128


@functools.lru_cache(maxsize=None)
def _make_sc_kernel(B, K, D, V):
    RB = B // NW
    NBUF = 4
    assert B % NW == 0 and RB % NBUF == 0 and K % 8 == 0
    bases = list(range(0, K, L))
    if bases[-1] + L > K:
        bases[-1] = K - L

    mesh = plsc.VectorSubcoreMesh(core_axis_name="c", subcore_axis_name="s")

    @functools.partial(
        pl.kernel,
        out_type=jax.ShapeDtypeStruct((B * K,), jnp.float32),
        mesh=mesh,
        compiler_params=pltpu.CompilerParams(
            use_tc_tiling_on_sc=False, needs_layout_passes=False),
        scratch_types=[
            pltpu.VMEM((RB * K,), jnp.int32),
            pltpu.VMEM((RB, D), jnp.float32),
            [pltpu.VMEM((K, D), jnp.float32) for _ in range(NBUF)],
            pltpu.VMEM((RB * K,), jnp.float32),
            [pltpu.SemaphoreType.DMA for _ in range(NBUF)],
        ],
    )
    def sc_kernel(z_hbm, idx_hbm, table_hbm, out_hbm,
                  idx_v, z_v, rows, out_v, sems):
        wid = lax.axis_index("s") * NC + lax.axis_index("c")
        base = wid * RB
        pltpu.sync_copy(idx_hbm.at[pl.ds(base * K, RB * K)], idx_v)
        pltpu.sync_copy(z_hbm.at[pl.ds(base, RB)], z_v)

        def issue(i, rows_buf, sem):
            pltpu.async_copy(
                table_hbm.at[idx_v.at[pl.ds(i * K, K)]], rows_buf, sem)

        def wait(rows_buf, sem):
            pltpu.make_async_copy(
                table_hbm.at[pl.ds(0, K)], rows_buf, sem).wait()

        iota = lax.iota(jnp.int32, L)

        def compute(i, rows_buf):
            ivec = jnp.broadcast_to(i, (L,))
            nh = len(bases) // 2 + 1
            for half in (bases[:nh], bases[nh:]):
                def rbody(r, accs, half=half):
                    rot = (iota + r) & (L - 1)
                    new = list(accs)
                    for dg in range(D // L):
                        dvec = dg * L + rot
                        m = plsc.load_gather(z_v, [ivec, dvec])
                        for gi, kb in enumerate(half):
                            col = plsc.load_gather(
                                rows_buf, [kb + iota, dvec])
                            new[gi] = new[gi] + m * col
                    return tuple(new)
                accs = lax.fori_loop(
                    0, L, rbody,
                    tuple(jnp.zeros((L,), jnp.float32) for _ in half))
                for gi, kb in enumerate(half):
                    out_v[pl.ds(i * K + kb, L)] = accs[gi]

        for s in range(NBUF):
            issue(s, rows[s], sems[s])

        def body(j, carry):
            for s in range(NBUF):
                i = NBUF * j + s
                wait(rows[s], sems[s])
                compute(i, rows[s])

                @pl.when(i + NBUF < RB)
                def _(s=s, i=i):
                    issue(i + NBUF, rows[s], sems[s])
            return carry

        lax.fori_loop(0, RB // NBUF, body, 0)
        pltpu.sync_copy(out_v, out_hbm.at[pl.ds(base * K, RB * K)])

    return sc_kernel


def kernel(z_B1D, cand_idx_BK, id_embed):
    B, _, D = z_B1D.shape
    K = cand_idx_BK.shape[1]
    V = id_embed.shape[0]
    z = z_B1D.reshape(B, D)
    idx = cand_idx_BK.reshape(-1)
    if idx.dtype != jnp.int32:
        idx = idx.astype(jnp.int32)
    table = jnp.pad(id_embed, ((0, 0), (0, TD - D))).reshape(2 * V, D)
    idx2 = idx * 2
    out = _make_sc_kernel(B, K, D, V)(z, idx2, table)
    return out.reshape(B, K)
